# baseline (speedup 1.0000x reference)
import functools

import jax
import jax.numpy as jnp
from jax.experimental import pallas as pl
from jax.experimental.pallas import tpu as pltpu

_BM = 512


def _body(hf, fr_ref, f2h_l_ref, f2h_r_ref, w_ref, b_ref, out_ref, fr_scratch):
    @pl.when(pl.program_id(0) == 0)
    def _():
        fr = jax.lax.dot_general(
            fr_ref[...], w_ref[...],
            dimension_numbers=(((1,), (1,)), ((), ())),
            preferred_element_type=jnp.float32,
        )
        fr_scratch[...] = (fr + b_ref[...]).astype(jnp.bfloat16)

    acc = jnp.dot(f2h_l_ref[...].astype(jnp.bfloat16), fr_scratch[:hf, :],
                  preferred_element_type=jnp.float32)
    acc += jnp.dot(f2h_r_ref[...].astype(jnp.bfloat16), fr_scratch[hf:, :],
                   preferred_element_type=jnp.float32)
    out_ref[...] = jnp.maximum(acc, 0.0)


@functools.partial(jax.jit, static_argnames=())
def kernel(fact_relations, fact2head, fact2tail, W, b):
    del fact2tail
    N, F = fact2head.shape
    H = fact_relations.shape[1]
    hf = F // 2
    b2 = b.reshape(1, H)

    grid = (N // _BM,)
    out = pl.pallas_call(
        functools.partial(_body, hf),
        grid=grid,
        in_specs=[
            pl.BlockSpec((F, H), lambda i: (0, 0)),
            pl.BlockSpec((_BM, hf), lambda i: (i, 0)),
            pl.BlockSpec((_BM, hf), lambda i: (i, 1)),
            pl.BlockSpec((H, H), lambda i: (0, 0)),
            pl.BlockSpec((1, H), lambda i: (0, 0)),
        ],
        out_specs=pl.BlockSpec((_BM, H), lambda i: (i, 0)),
        out_shape=jax.ShapeDtypeStruct((N, H), jnp.float32),
        scratch_shapes=[pltpu.VMEM((F, H), jnp.bfloat16)],
    )(fact_relations, fact2head, fact2head, W, b2)
    return out


# speedup vs baseline: 1.0040x; 1.0040x over previous
import functools

import jax
import jax.numpy as jnp
from jax.experimental import pallas as pl
from jax.experimental.pallas import tpu as pltpu

_BM = 512


def _body(fr_ref, f2h_t_ref, f2h_b_ref, w_ref, b_ref, out_ref, fr_scratch):
    @pl.when(pl.program_id(0) == 0)
    def _():
        fr = jax.lax.dot_general(
            fr_ref[...], w_ref[...],
            dimension_numbers=(((1,), (1,)), ((), ())),
            preferred_element_type=jnp.float32,
        )
        fr_scratch[...] = (fr + b_ref[...]).astype(jnp.bfloat16)

    hm = _BM // 2
    fr = fr_scratch[...]
    acc_t = jnp.dot(f2h_t_ref[...].astype(jnp.bfloat16), fr,
                    preferred_element_type=jnp.float32)
    out_ref[:hm, :] = jnp.maximum(acc_t, 0.0)
    acc_b = jnp.dot(f2h_b_ref[...].astype(jnp.bfloat16), fr,
                    preferred_element_type=jnp.float32)
    out_ref[hm:, :] = jnp.maximum(acc_b, 0.0)


@functools.partial(jax.jit, static_argnames=())
def kernel(fact_relations, fact2head, fact2tail, W, b):
    del fact2tail
    N, F = fact2head.shape
    H = fact_relations.shape[1]
    hm = _BM // 2
    b2 = b.reshape(1, H)

    grid = (N // _BM,)
    out = pl.pallas_call(
        _body,
        grid=grid,
        in_specs=[
            pl.BlockSpec((F, H), lambda i: (0, 0)),
            pl.BlockSpec((hm, F), lambda i: (2 * i, 0)),
            pl.BlockSpec((hm, F), lambda i: (2 * i + 1, 0)),
            pl.BlockSpec((H, H), lambda i: (0, 0)),
            pl.BlockSpec((1, H), lambda i: (0, 0)),
        ],
        out_specs=pl.BlockSpec((_BM, H), lambda i: (i, 0)),
        out_shape=jax.ShapeDtypeStruct((N, H), jnp.float32),
        scratch_shapes=[pltpu.VMEM((F, H), jnp.bfloat16)],
    )(fact_relations, fact2head, fact2head, W, b2)
    return out


# speedup vs baseline: 1.0548x; 1.0507x over previous
import functools

import jax
import jax.numpy as jnp
from jax.experimental import pallas as pl
from jax.experimental.pallas import tpu as pltpu

_BM = 512


def _body(fr_ref, f2h_ref, w_ref, b_ref, out_ref, fr_scratch):
    @pl.when(pl.program_id(0) == 0)
    def _():
        fr = jax.lax.dot_general(
            fr_ref[...], w_ref[...],
            dimension_numbers=(((1,), (1,)), ((), ())),
            preferred_element_type=jnp.float32,
        )
        fr_scratch[...] = (fr + b_ref[...]).astype(jnp.bfloat16)

    acc = jnp.dot(f2h_ref[...].astype(jnp.bfloat16), fr_scratch[...],
                  preferred_element_type=jnp.float32)
    out_ref[...] = jnp.maximum(acc, 0.0)


@functools.partial(jax.jit, static_argnames=())
def kernel(fact_relations, fact2head, fact2tail, W, b):
    del fact2tail
    N, F = fact2head.shape
    H = fact_relations.shape[1]
    b2 = b.reshape(1, H)

    grid = (N // _BM,)
    out = pl.pallas_call(
        _body,
        grid=grid,
        in_specs=[
            pl.BlockSpec((F, H), lambda i: (0, 0)),
            pl.BlockSpec((_BM, F), lambda i: (i, 0)),
            pl.BlockSpec((H, H), lambda i: (0, 0)),
            pl.BlockSpec((1, H), lambda i: (0, 0)),
        ],
        out_specs=pl.BlockSpec((_BM, H), lambda i: (i, 0)),
        out_shape=jax.ShapeDtypeStruct((N, H), jnp.float32),
        scratch_shapes=[pltpu.VMEM((F, H), jnp.bfloat16)],
    )(fact_relations, fact2head, W, b2)
    return out
